# baseline (speedup 1.0000x reference)
import jax
import jax.numpy as jnp
from jax.experimental import pallas as pl
from jax.experimental.pallas import tpu as pltpu

DIM_ = 1024
ROW_BLOCK = 512
NSPLIT = 2
STEPS_PER = 3072 // NSPLIT


def _build_m_kernel(pairs_i_ref, pairs_j_ref, angles_ref, m_ref, cs_ref):
    h = pl.program_id(0)
    base = h * STEPS_PER

    a = angles_ref[...]
    cs_ref[:, 0:1] = jnp.cos(a)
    cs_ref[:, 1:2] = jnp.sin(a)

    p_ids = jax.lax.broadcasted_iota(jnp.int32, (DIM_ * 8, 128), 0)
    l_ids = jax.lax.broadcasted_iota(jnp.int32, (DIM_ * 8, 128), 1)
    logical_col = 128 * (p_ids % 8) + l_ids
    m_ref[0, :, :] = jnp.where(logical_col == p_ids // 8, 1.0, 0.0).astype(
        jnp.float32
    )

    def body(t, _):
        tg = base + t
        ib = pairs_i_ref[tg] * 8
        jb = pairs_j_ref[tg] * 8
        tm = jax.lax.rem(tg, DIM_)
        c = cs_ref[pl.ds(tm, 1), 0:1]
        s = cs_ref[pl.ds(tm, 1), 1:2]
        mi = m_ref[0, pl.ds(ib, 8), :]
        mj = m_ref[0, pl.ds(jb, 8), :]
        m_ref[0, pl.ds(ib, 8), :] = c * mi - s * mj
        m_ref[0, pl.ds(jb, 8), :] = s * mi + c * mj
        return 0

    jax.lax.fori_loop(0, STEPS_PER, body, 0, unroll=8)


def _combine_kernel(a_ref, b_ref, o_ref):
    o_ref[...] = jnp.dot(
        b_ref[...], a_ref[...], preferred_element_type=jnp.float32
    )


def _matmul_kernel(x_ref, m_ref, o_ref):
    o_ref[...] = jax.lax.dot_general(
        x_ref[...], m_ref[...],
        dimension_numbers=(((1,), (1,)), ((), ())),
        preferred_element_type=jnp.float32,
    )


def kernel(x, angles, pairs_i, pairs_j):
    dim = angles.shape[0]
    x2d = x.reshape(-1, dim).astype(jnp.float32)
    n_rows = x2d.shape[0]

    m8 = pl.pallas_call(
        _build_m_kernel,
        out_shape=jax.ShapeDtypeStruct((NSPLIT, dim * 8, 128), jnp.float32),
        grid=(NSPLIT,),
        in_specs=[
            pl.BlockSpec(memory_space=pltpu.SMEM),
            pl.BlockSpec(memory_space=pltpu.SMEM),
            pl.BlockSpec((dim, 1), lambda h: (0, 0)),
        ],
        out_specs=pl.BlockSpec((1, dim * 8, 128), lambda h: (h, 0, 0)),
        scratch_shapes=[pltpu.VMEM((dim, 2), jnp.float32)],
        compiler_params=pltpu.CompilerParams(
            dimension_semantics=("parallel",),
        ),
    )(pairs_i, pairs_j, angles.reshape(dim, 1).astype(jnp.float32))
    halves = m8.reshape(NSPLIT, dim, dim)

    combine = pl.pallas_call(
        _combine_kernel,
        out_shape=jax.ShapeDtypeStruct((dim, dim), jnp.float32),
    )
    c = halves[0]
    for h in range(1, NSPLIT):
        c = combine(c, halves[h])

    grid = (n_rows // ROW_BLOCK,)
    y2d = pl.pallas_call(
        _matmul_kernel,
        out_shape=jax.ShapeDtypeStruct((n_rows, dim), jnp.float32),
        grid=grid,
        in_specs=[
            pl.BlockSpec((ROW_BLOCK, dim), lambda r: (r, 0)),
            pl.BlockSpec((dim, dim), lambda r: (0, 0)),
        ],
        out_specs=pl.BlockSpec((ROW_BLOCK, dim), lambda r: (r, 0)),
    )(x2d, c)

    return y2d.reshape(x.shape).astype(x.dtype)
